# baseline (speedup 1.0000x reference)
import functools

import jax
import jax.numpy as jnp
from jax import lax
from jax.experimental import pallas as pl
from jax.experimental.pallas import tpu as pltpu
from jax.experimental.pallas import tpu_sc as plsc

_N = 8388608
_NC = 2
_NS = 16
_NW = _NC * _NS
_L = 16
_PER_W = _N // _NW
_CHUNK = 16384
_NCHUNK = _PER_W // _CHUNK
_VPC = _CHUNK // _L

_C0 = 1.1447097560735031e-05
_C1 = 0.9991664010110692
_C2 = -0.48969909032083947
_C3 = 0.28382318306531834
_C4 = -0.1299571976582333
_C5 = 0.029808765243435193


def _partials_kernel(prob_hbm, labels_hbm, out_hbm, pbuf, lbuf, stage,
                     sem0, sem1):
    cid = lax.axis_index("c")
    sid = lax.axis_index("s")
    wid = sid * _NC + cid
    base = wid * _PER_W
    sems = (sem0, sem1)

    def start(chunk_idx, buf):
        off = base + chunk_idx * _CHUNK
        hp = pltpu.async_copy(prob_hbm.at[pl.ds(off, _CHUNK)],
                              pbuf.at[buf], sems[buf])
        hl = pltpu.async_copy(labels_hbm.at[pl.ds(off, _CHUNK)],
                              lbuf.at[buf], sems[buf])
        return hp, hl

    def make_inner(buf):
        def inner(j, accs):
            s_tot, s_pos, c_pos, c_pl, c_nl, c_pp, c_np = accs
            x = pbuf[buf, pl.ds(j * _L, _L)]
            t = lbuf[buf, pl.ds(j * _L, _L)]
            e = jnp.exp(-jnp.abs(x))
            q = _C5
            q = q * e + _C4
            q = q * e + _C3
            q = q * e + _C2
            q = q * e + _C1
            q = q * e + _C0
            sp = jnp.maximum(x, 0.0) + q
            a = jnp.minimum(sp, 100.0)
            b = jnp.minimum(sp - x, 100.0)
            bce = a + t * (b - a)
            posm = t > 0.5
            predf = jnp.where(x > 0.0, 1.0, 0.0)
            s2 = predf + t
            s_tot = s_tot + bce
            s_pos = s_pos + jnp.where(posm, bce, 0.0)
            c_pos = c_pos + jnp.where(posm, 1.0, 0.0)
            c_pl = c_pl + jnp.where(t == 1.0, 1.0, 0.0)
            c_nl = c_nl + jnp.where(t == 0.0, 1.0, 0.0)
            c_pp = c_pp + jnp.where(s2 == 2.0, 1.0, 0.0)
            c_np = c_np + jnp.where(s2 == 0.0, 1.0, 0.0)
            return (s_tot, s_pos, c_pos, c_pl, c_nl, c_pp, c_np)
        return inner

    z = jnp.zeros((_L,), jnp.float32)
    accs = (z, z, z, z, z, z, z)

    handles = [None, None]
    handles[0] = start(0, 0)
    for i in range(_NCHUNK):
        buf = i % 2
        if i + 1 < _NCHUNK:
            handles[1 - buf] = start(i + 1, 1 - buf)
        hp, hl = handles[buf]
        hp.wait()
        hl.wait()
        accs = lax.fori_loop(0, _VPC, make_inner(buf), accs)

    for k in range(7):
        stage[k] = accs[k]
    pltpu.sync_copy(stage, out_hbm.at[wid])


@jax.jit
def kernel(prob, labels):
    mesh = plsc.VectorSubcoreMesh(core_axis_name="c", subcore_axis_name="s")
    partials = pl.kernel(
        _partials_kernel,
        mesh=mesh,
        out_type=jax.ShapeDtypeStruct((_NW, 7, _L), jnp.float32),
        scratch_types=[
            pltpu.VMEM((2, _CHUNK), jnp.float32),
            pltpu.VMEM((2, _CHUNK), jnp.float32),
            pltpu.VMEM((7, _L), jnp.float32),
            pltpu.SemaphoreType.DMA,
            pltpu.SemaphoreType.DMA,
        ],
    )(prob, labels)

    sums = jnp.sum(partials, axis=(0, 2))
    s_tot = sums[0]
    s_pos = sums[1]
    c_pos = sums[2]
    c_pl = sums[3]
    c_nl = sums[4]
    c_pp = sums[5]
    c_np = sums[6]
    s_neg = s_tot - s_pos
    c_neg = jnp.float32(_N) - c_pos
    pos_loss = 0.5 * s_pos / jnp.maximum(c_pos, 1.0)
    neg_loss = 0.5 * s_neg / jnp.maximum(c_neg, 1.0)
    classify_loss = (jnp.where(c_pos > 0.0, pos_loss, 0.0)
                     + jnp.where(c_neg > 0.0, neg_loss, 0.0))
    return (classify_loss,
            c_pp.astype(jnp.int32),
            c_pl.astype(jnp.int32),
            c_np.astype(jnp.int32),
            c_nl.astype(jnp.int32))
